# Initial kernel scaffold; baseline (speedup 1.0000x reference)
import jax
import jax.numpy as jnp
from jax.experimental import pallas as pl


def kernel(hidden_states, gate_w, w1, w2, w3, sg, su, sd):
    raise NotImplementedError("write your pallas kernel here")


# speedup vs baseline: 1.6761x; 1.6761x over previous
import functools

import jax
import jax.numpy as jnp
from jax.experimental import pallas as pl
from jax.experimental.pallas import tpu as pltpu

_TB = 512


def _silu(v):
    return v * jax.lax.logistic(v)


def _gate_body(x_ref, gate_ref, comb_ref):
    x = x_ref[...]
    logits = jnp.dot(x, gate_ref[...].T,
                     preferred_element_type=jnp.float32)
    m = jnp.max(logits, axis=1, keepdims=True)
    p = jnp.exp(logits - m)
    s = p / jnp.sum(p, axis=1, keepdims=True)
    n_e = s.shape[1]
    lane = jax.lax.broadcasted_iota(jnp.int32, s.shape, 1)
    m1 = jnp.max(s, axis=1, keepdims=True)
    i1 = jnp.min(jnp.where(s == m1, lane, n_e), axis=1, keepdims=True)
    sel1 = lane == i1
    s2 = jnp.where(sel1, -jnp.inf, s)
    m2 = jnp.max(s2, axis=1, keepdims=True)
    i2 = jnp.min(jnp.where(s2 == m2, lane, n_e), axis=1, keepdims=True)
    sel2 = lane == i2
    denom = m1 + m2 + 1e-20
    comb_ref[...] = (jnp.where(sel1, m1, 0.0)
                     + jnp.where(sel2, m2, 0.0)) / denom


def _shared_body(x_ref, sg_ref, su_ref, sd_ref, out_ref):
    x = x_ref[...]
    g = jnp.dot(x, sg_ref[...].T, preferred_element_type=jnp.float32)
    u = jnp.dot(x, su_ref[...].T, preferred_element_type=jnp.float32)
    h = _silu(g) * u
    out_ref[...] = jnp.dot(h, sd_ref[...].T, preferred_element_type=jnp.float32)


def _routed_body(x_ref, w1_ref, w3_ref, w2_ref, comb_ref, shared_ref, out_ref):
    e = pl.program_id(1)
    x = x_ref[...]
    g = jnp.dot(x, w1_ref[0].T, preferred_element_type=jnp.float32)
    u = jnp.dot(x, w3_ref[0].T, preferred_element_type=jnp.float32)
    h = _silu(g) * u
    y = jnp.dot(h, w2_ref[0].T, preferred_element_type=jnp.float32)
    c = comb_ref[...]
    lane = jax.lax.broadcasted_iota(jnp.int32, c.shape, 1)
    w_e = jnp.sum(jnp.where(lane == e, c, 0.0), axis=1, keepdims=True)

    @pl.when(e == 0)
    def _init():
        out_ref[...] = shared_ref[...] + y * w_e

    @pl.when(e != 0)
    def _acc():
        out_ref[...] += y * w_e


def kernel(hidden_states, gate_w, w1, w2, w3, sg, su, sd):
    b, s, d = hidden_states.shape
    x = hidden_states.reshape(-1, d).astype(jnp.float32)
    t = x.shape[0]
    n_exp, d_ff, _ = w1.shape
    sf = sg.shape[0]
    n_tb = t // _TB

    comb = pl.pallas_call(
        _gate_body,
        in_specs=[
            pl.BlockSpec((t, d), lambda: (0, 0)),
            pl.BlockSpec(gate_w.shape, lambda: (0, 0)),
        ],
        out_specs=pl.BlockSpec((t, n_exp), lambda: (0, 0)),
        out_shape=jax.ShapeDtypeStruct((t, n_exp), jnp.float32),
    )(x, gate_w)

    shared = pl.pallas_call(
        _shared_body,
        grid=(n_tb,),
        in_specs=[
            pl.BlockSpec((_TB, d), lambda i: (i, 0)),
            pl.BlockSpec((sf, d), lambda i: (0, 0)),
            pl.BlockSpec((sf, d), lambda i: (0, 0)),
            pl.BlockSpec((d, sf), lambda i: (0, 0)),
        ],
        out_specs=pl.BlockSpec((_TB, d), lambda i: (i, 0)),
        out_shape=jax.ShapeDtypeStruct((t, d), jnp.float32),
    )(x, sg, su, sd)

    out = pl.pallas_call(
        _routed_body,
        grid=(n_tb, n_exp),
        in_specs=[
            pl.BlockSpec((_TB, d), lambda i, e: (i, 0)),
            pl.BlockSpec((1, d_ff, d), lambda i, e: (e, 0, 0)),
            pl.BlockSpec((1, d_ff, d), lambda i, e: (e, 0, 0)),
            pl.BlockSpec((1, d, d_ff), lambda i, e: (e, 0, 0)),
            pl.BlockSpec((_TB, n_exp), lambda i, e: (i, 0)),
            pl.BlockSpec((_TB, d), lambda i, e: (i, 0)),
        ],
        out_specs=pl.BlockSpec((_TB, d), lambda i, e: (i, 0)),
        out_shape=jax.ShapeDtypeStruct((t, d), jnp.float32),
        compiler_params=pltpu.CompilerParams(
            dimension_semantics=("arbitrary", "arbitrary"),
        ),
    )(x, w1, w3, w2, comb, shared)

    return out.reshape(b, s, d).astype(hidden_states.dtype)
